# baseline (speedup 1.0000x reference)
import functools

import jax
import jax.numpy as jnp
from jax.experimental import pallas as pl
from jax.experimental.pallas import tpu as pltpu

N = 10000
D = 128
F1 = 20
F2 = 20
BR1 = 400
BR23 = 400


def _pass0_body(x_ref, w1_ref, s1_ref):
    s1_ref[...] = jnp.dot(
        x_ref[...], w1_ref[...], preferred_element_type=jnp.float32
    ).astype(jnp.bfloat16)


def _pass1_body(adj_ref, s1_ref, b1_ref, w2_ref, adjb_ref, s2_ref):
    a = adj_ref[...].astype(jnp.bfloat16)
    adjb_ref[...] = a
    h = jnp.dot(a, s1_ref[...], preferred_element_type=jnp.float32)
    h = jnp.maximum(h + b1_ref[...], 0.0)
    s2_ref[...] = jnp.dot(
        h, w2_ref[...], preferred_element_type=jnp.float32
    ).astype(jnp.bfloat16)


def _pass23_body(adjb_ref, s2_ref, b2_ref, w3_ref, b3_ref, x_ref,
                 out_ref, s3_scr):
    p = pl.program_id(0)
    i = pl.program_id(1)

    @pl.when(p == 0)
    def _():
        h = jnp.dot(adjb_ref[...], s2_ref[...],
                    preferred_element_type=jnp.float32)
        h = jnp.maximum(h + b2_ref[...], 0.0)
        s3_scr[pl.ds(i * BR23, BR23), :] = jnp.dot(
            h, w3_ref[...], preferred_element_type=jnp.float32
        ).astype(jnp.bfloat16)

    @pl.when(p == 1)
    def _():
        h = jnp.dot(adjb_ref[...], s3_scr[...],
                    preferred_element_type=jnp.float32)
        h = jnp.maximum(h + b3_ref[...], 0.0)
        out_ref[...] = jnp.maximum(h + x_ref[...], 0.0)


@functools.partial(jax.jit, static_argnames=())
def kernel(x, adj, W1, b1, W2, b2, W3, b3):
    b1r = b1.reshape(1, F1)
    b2r = b2.reshape(1, F2)
    b3r = b3.reshape(1, D)

    s1 = pl.pallas_call(
        _pass0_body,
        out_shape=jax.ShapeDtypeStruct((N, F1), jnp.bfloat16),
    )(x, W1)

    adj_bf16, s2 = pl.pallas_call(
        _pass1_body,
        grid=(N // BR1,),
        in_specs=[
            pl.BlockSpec((BR1, N), lambda i: (i, 0)),
            pl.BlockSpec((N, F1), lambda i: (0, 0)),
            pl.BlockSpec((1, F1), lambda i: (0, 0)),
            pl.BlockSpec((F1, F2), lambda i: (0, 0)),
        ],
        out_specs=[
            pl.BlockSpec((BR1, N), lambda i: (i, 0)),
            pl.BlockSpec((BR1, F2), lambda i: (i, 0)),
        ],
        out_shape=[
            jax.ShapeDtypeStruct((N, N), jnp.bfloat16),
            jax.ShapeDtypeStruct((N, F2), jnp.bfloat16),
        ],
    )(adj, s1, b1r, W2)

    out = pl.pallas_call(
        _pass23_body,
        grid=(2, N // BR23),
        in_specs=[
            pl.BlockSpec((BR23, N), lambda p, i: (i, 0)),
            pl.BlockSpec((N, F2), lambda p, i: (0, 0)),
            pl.BlockSpec((1, F2), lambda p, i: (0, 0)),
            pl.BlockSpec((F2, D), lambda p, i: (0, 0)),
            pl.BlockSpec((1, D), lambda p, i: (0, 0)),
            pl.BlockSpec((BR23, D), lambda p, i: (p * i, 0)),
        ],
        out_specs=pl.BlockSpec((BR23, D), lambda p, i: (p * i, 0)),
        out_shape=jax.ShapeDtypeStruct((N, D), jnp.float32),
        scratch_shapes=[pltpu.VMEM((N, D), jnp.bfloat16)],
    )(adj_bf16, s2, b2r, W3, b3r, x)

    return out


# speedup vs baseline: 1.0596x; 1.0596x over previous
import functools

import jax
import jax.numpy as jnp
from jax.experimental import pallas as pl
from jax.experimental.pallas import tpu as pltpu

N = 10000
D = 128
F1 = 20
F2 = 20
BR1 = 400
BR23 = 1000


def _pass0_body(x_ref, w1_ref, s1_ref):
    s1_ref[...] = jnp.dot(
        x_ref[...], w1_ref[...], preferred_element_type=jnp.float32
    ).astype(jnp.bfloat16)


def _pass1_body(adj_ref, x_ref, w1_ref, b1_ref, w2_ref, adjb_ref, s2_ref,
                s1_scr):
    @pl.when(pl.program_id(0) == 0)
    def _():
        s1_scr[...] = jnp.dot(
            x_ref[...], w1_ref[...], preferred_element_type=jnp.float32
        ).astype(jnp.bfloat16)

    a = adj_ref[...].astype(jnp.bfloat16)
    adjb_ref[...] = a
    h = jnp.dot(a, s1_scr[...], preferred_element_type=jnp.float32)
    h = jnp.maximum(h + b1_ref[...], 0.0)
    s2_ref[...] = jnp.dot(
        h, w2_ref[...], preferred_element_type=jnp.float32
    ).astype(jnp.bfloat16)


def _pass23_body(adjb_ref, s2_ref, b2_ref, w3_ref, b3_ref, x_ref,
                 out_ref, s3_scr):
    p = pl.program_id(0)
    i = pl.program_id(1)

    @pl.when(p == 0)
    def _():
        h = jnp.dot(adjb_ref[...], s2_ref[...],
                    preferred_element_type=jnp.float32)
        h = jnp.maximum(h + b2_ref[...], 0.0)
        s3_scr[pl.ds(i * BR23, BR23), :] = jnp.dot(
            h, w3_ref[...], preferred_element_type=jnp.float32
        ).astype(jnp.bfloat16)

    @pl.when(p == 1)
    def _():
        h = jnp.dot(adjb_ref[...], s3_scr[...],
                    preferred_element_type=jnp.float32)
        h = jnp.maximum(h + b3_ref[...], 0.0)
        out_ref[...] = jnp.maximum(h + x_ref[...], 0.0)


@functools.partial(jax.jit, static_argnames=())
def kernel(x, adj, W1, b1, W2, b2, W3, b3):
    b1r = b1.reshape(1, F1)
    b2r = b2.reshape(1, F2)
    b3r = b3.reshape(1, D)

    adj_bf16, s2 = pl.pallas_call(
        _pass1_body,
        grid=(N // BR1,),
        in_specs=[
            pl.BlockSpec((BR1, N), lambda i: (i, 0)),
            pl.BlockSpec((N, D), lambda i: (0, 0)),
            pl.BlockSpec((D, F1), lambda i: (0, 0)),
            pl.BlockSpec((1, F1), lambda i: (0, 0)),
            pl.BlockSpec((F1, F2), lambda i: (0, 0)),
        ],
        out_specs=[
            pl.BlockSpec((BR1, N), lambda i: (i, 0)),
            pl.BlockSpec((BR1, F2), lambda i: (i, 0)),
        ],
        out_shape=[
            jax.ShapeDtypeStruct((N, N), jnp.bfloat16),
            jax.ShapeDtypeStruct((N, F2), jnp.bfloat16),
        ],
        scratch_shapes=[pltpu.VMEM((N, F1), jnp.bfloat16)],
    )(adj, x, W1, b1r, W2)

    out = pl.pallas_call(
        _pass23_body,
        grid=(2, N // BR23),
        in_specs=[
            pl.BlockSpec((BR23, N), lambda p, i: (i, 0)),
            pl.BlockSpec((N, F2), lambda p, i: (0, 0)),
            pl.BlockSpec((1, F2), lambda p, i: (0, 0)),
            pl.BlockSpec((F2, D), lambda p, i: (0, 0)),
            pl.BlockSpec((1, D), lambda p, i: (0, 0)),
            pl.BlockSpec((BR23, D), lambda p, i: (p * i, 0)),
        ],
        out_specs=pl.BlockSpec((BR23, D), lambda p, i: (p * i, 0)),
        out_shape=jax.ShapeDtypeStruct((N, D), jnp.float32),
        scratch_shapes=[pltpu.VMEM((N, D), jnp.bfloat16)],
    )(adj_bf16, s2, b2r, W3, b3r, x)

    return out


# speedup vs baseline: 1.0602x; 1.0006x over previous
import functools

import jax
import jax.numpy as jnp
from jax.experimental import pallas as pl
from jax.experimental.pallas import tpu as pltpu

N = 10000
D = 128
F1 = 20
F2 = 20
BR1 = 400
BR23 = 1000


def _pass1_body(adj_ref, x_ref, w1_ref, b1_ref, w2_ref, adjb_ref, s2_ref,
                s1_scr):
    @pl.when(pl.program_id(0) == 0)
    def _():
        s1_scr[...] = jnp.dot(
            x_ref[...], w1_ref[...], preferred_element_type=jnp.float32
        ).astype(jnp.bfloat16)

    a = adj_ref[...].astype(jnp.bfloat16)
    adjb_ref[...] = a
    h = jnp.dot(a, s1_scr[...], preferred_element_type=jnp.float32)
    h = jnp.maximum(h + b1_ref[...], 0.0)
    s2_ref[...] = jnp.dot(
        h, w2_ref[...], preferred_element_type=jnp.float32
    ).astype(jnp.bfloat16)


def _pass23_body(adjb_ref, s2_ref, b2_ref, w3_ref, b3_ref, x_ref,
                 out_ref, s3_scr):
    p = pl.program_id(0)
    i = pl.program_id(1)

    @pl.when(p == 0)
    def _():
        h = jnp.dot(adjb_ref[...], s2_ref[...],
                    preferred_element_type=jnp.float32)
        h = jnp.maximum(h + b2_ref[...], 0.0)
        s3_scr[pl.ds(i * BR23, BR23), :] = jnp.dot(
            h, w3_ref[...], preferred_element_type=jnp.float32
        ).astype(jnp.bfloat16)

    @pl.when(p == 1)
    def _():
        h = jnp.dot(adjb_ref[...], s3_scr[...],
                    preferred_element_type=jnp.float32)
        h = jnp.maximum(h + b3_ref[...], 0.0)
        out_ref[...] = jnp.maximum(h + x_ref[...], 0.0)


@functools.partial(jax.jit, static_argnames=())
def kernel(x, adj, W1, b1, W2, b2, W3, b3):
    b1r = b1.reshape(1, F1)
    b2r = b2.reshape(1, F2)
    b3r = b3.reshape(1, D)

    adj_bf16, s2 = pl.pallas_call(
        _pass1_body,
        grid=(N // BR1,),
        in_specs=[
            pl.BlockSpec((BR1, N), lambda i: (i, 0)),
            pl.BlockSpec((N, D), lambda i: (0, 0)),
            pl.BlockSpec((D, F1), lambda i: (0, 0)),
            pl.BlockSpec((1, F1), lambda i: (0, 0)),
            pl.BlockSpec((F1, F2), lambda i: (0, 0)),
        ],
        out_specs=[
            pl.BlockSpec((BR1, N), lambda i: (i, 0)),
            pl.BlockSpec((BR1, F2), lambda i: (i, 0)),
        ],
        out_shape=[
            jax.ShapeDtypeStruct((N, N), jnp.bfloat16),
            jax.ShapeDtypeStruct((N, F2), jnp.bfloat16),
        ],
        scratch_shapes=[pltpu.VMEM((N, F1), jnp.bfloat16)],
    )(adj, x, W1, b1r, W2)

    out = pl.pallas_call(
        _pass23_body,
        grid=(2, N // BR23),
        in_specs=[
            pl.BlockSpec((BR23, N), lambda p, i: (i, 0)),
            pl.BlockSpec((N, F2), lambda p, i: (0, 0)),
            pl.BlockSpec((1, F2), lambda p, i: (0, 0)),
            pl.BlockSpec((F2, D), lambda p, i: (0, 0)),
            pl.BlockSpec((1, D), lambda p, i: (0, 0)),
            pl.BlockSpec((BR23, D), lambda p, i: (p * i, 0)),
        ],
        out_specs=pl.BlockSpec((BR23, D), lambda p, i: (p * i, 0)),
        out_shape=jax.ShapeDtypeStruct((N, D), jnp.float32),
        scratch_shapes=[pltpu.VMEM((N, D), jnp.bfloat16)],
    )(adj_bf16, s2, b2r, W3, b3r, x)

    return out


# speedup vs baseline: 1.2452x; 1.1745x over previous
import functools

import jax
import jax.numpy as jnp
from jax.experimental import pallas as pl
from jax.experimental.pallas import tpu as pltpu

N = 10000
D = 128
F1 = 20
F2 = 20
BR1 = 400
BR23 = 1000


def _pass1_body(adj_ref, x_ref, w1_ref, b1_ref, w2_ref, adjb_ref, s2_ref,
                s1_scr):
    @pl.when(pl.program_id(0) == 0)
    def _():
        s1_scr[...] = jnp.dot(
            x_ref[...], w1_ref[...], preferred_element_type=jnp.float32
        ).astype(jnp.bfloat16)

    a = adj_ref[...].astype(jnp.bfloat16)
    adjb_ref[...] = adj_ref[...].astype(jnp.float8_e4m3fn)
    h = jnp.dot(a, s1_scr[...], preferred_element_type=jnp.float32)
    h = jnp.maximum(h + b1_ref[...], 0.0)
    s2_ref[...] = jnp.dot(
        h, w2_ref[...], preferred_element_type=jnp.float32
    ).astype(jnp.bfloat16)


def _pass23_body(adjb_ref, s2_ref, b2_ref, w3_ref, b3_ref, x_ref,
                 out_ref, s3_scr):
    p = pl.program_id(0)
    i = pl.program_id(1)

    @pl.when(p == 0)
    def _():
        h = jnp.dot(adjb_ref[...].astype(jnp.bfloat16), s2_ref[...],
                    preferred_element_type=jnp.float32)
        h = jnp.maximum(h + b2_ref[...], 0.0)
        s3_scr[pl.ds(i * BR23, BR23), :] = jnp.dot(
            h, w3_ref[...], preferred_element_type=jnp.float32
        ).astype(jnp.bfloat16)

    @pl.when(p == 1)
    def _():
        h = jnp.dot(adjb_ref[...].astype(jnp.bfloat16), s3_scr[...],
                    preferred_element_type=jnp.float32)
        h = jnp.maximum(h + b3_ref[...], 0.0)
        out_ref[...] = jnp.maximum(h + x_ref[...], 0.0)


@functools.partial(jax.jit, static_argnames=())
def kernel(x, adj, W1, b1, W2, b2, W3, b3):
    b1r = b1.reshape(1, F1)
    b2r = b2.reshape(1, F2)
    b3r = b3.reshape(1, D)

    adj_bf16, s2 = pl.pallas_call(
        _pass1_body,
        grid=(N // BR1,),
        in_specs=[
            pl.BlockSpec((BR1, N), lambda i: (i, 0)),
            pl.BlockSpec((N, D), lambda i: (0, 0)),
            pl.BlockSpec((D, F1), lambda i: (0, 0)),
            pl.BlockSpec((1, F1), lambda i: (0, 0)),
            pl.BlockSpec((F1, F2), lambda i: (0, 0)),
        ],
        out_specs=[
            pl.BlockSpec((BR1, N), lambda i: (i, 0)),
            pl.BlockSpec((BR1, F2), lambda i: (i, 0)),
        ],
        out_shape=[
            jax.ShapeDtypeStruct((N, N), jnp.float8_e4m3fn),
            jax.ShapeDtypeStruct((N, F2), jnp.bfloat16),
        ],
        scratch_shapes=[pltpu.VMEM((N, F1), jnp.bfloat16)],
    )(adj, x, W1, b1r, W2)

    out = pl.pallas_call(
        _pass23_body,
        grid=(2, N // BR23),
        in_specs=[
            pl.BlockSpec((BR23, N), lambda p, i: (i, 0)),
            pl.BlockSpec((N, F2), lambda p, i: (0, 0)),
            pl.BlockSpec((1, F2), lambda p, i: (0, 0)),
            pl.BlockSpec((F2, D), lambda p, i: (0, 0)),
            pl.BlockSpec((1, D), lambda p, i: (0, 0)),
            pl.BlockSpec((BR23, D), lambda p, i: (p * i, 0)),
        ],
        out_specs=pl.BlockSpec((BR23, D), lambda p, i: (p * i, 0)),
        out_shape=jax.ShapeDtypeStruct((N, D), jnp.float32),
        scratch_shapes=[pltpu.VMEM((N, D), jnp.bfloat16)],
    )(adj_bf16, s2, b2r, W3, b3r, x)

    return out
